# baseline (speedup 1.0000x reference)
import functools

import jax
import jax.numpy as jnp
from jax import lax
from jax.experimental import pallas as pl
from jax.experimental.pallas import tpu as pltpu
from jax.experimental.pallas import tpu_sc as plsc

_B = 16384
_D = 64
_NW = 32
_BPW = _B // _NW
_CH = 128
_NCH = _BPW // _CH
_W = 8
_L = 16

_mesh = plsc.VectorSubcoreMesh(core_axis_name="c", subcore_axis_name="s")


@functools.partial(
    pl.kernel,
    mesh=_mesh,
    compiler_params=pltpu.CompilerParams(
        use_tc_tiling_on_sc=True, needs_layout_passes=False),
    out_type=jax.ShapeDtypeStruct((_NW * _NCH, _CH, 2 * _D), jnp.float32),
    scratch_types=[
        pltpu.VMEM((_NCH, _CH), jnp.int32),
        pltpu.VMEM((_NCH, _CH, 2 * _D), jnp.float32),
        pltpu.SemaphoreType.DMA,
    ],
)
def _gather_z(hi1_hbm, emb_hbm, z_out, hi1_v, rows_v, sem):
    wid = lax.axis_index("s") * 2 + lax.axis_index("c")
    base = wid * _NCH
    pltpu.sync_copy(hi1_hbm.at[pl.ds(base, _NCH)], hi1_v)
    copies = [pltpu.async_copy(emb_hbm.at[hi1_v.at[j]], rows_v.at[j], sem)
              for j in range(_NCH)]
    for c in copies:
        c.wait()
    pltpu.sync_copy(rows_v, z_out.at[pl.ds(base, _NCH)])


@functools.partial(
    pl.kernel,
    mesh=_mesh,
    compiler_params=pltpu.CompilerParams(
        use_tc_tiling_on_sc=False, needs_layout_passes=False),
    out_type=[
        jax.ShapeDtypeStruct((_NW * _NCH, _CH), jnp.float32),
        jax.ShapeDtypeStruct((_NW * _NCH, _CH), jnp.float32),
    ],
    scratch_types=[
        pltpu.VMEM((_NCH, _CH), jnp.int32),
        pltpu.VMEM((_NCH, _CH), jnp.int32),
        pltpu.VMEM((_NCH, _CH, _W), jnp.float32),
        pltpu.VMEM((_NCH, _CH, _W), jnp.float32),
        pltpu.VMEM((_NCH, _CH), jnp.float32),
        pltpu.VMEM((_NCH, _CH), jnp.float32),
        pltpu.SemaphoreType.DMA,
    ],
)
def _gather_small(idx_hbm, hi3_hbm, inh_hbm, det_hbm,
                  inh_out, det_out,
                  idx_v, hi3_v, inh_rows, det_rows,
                  inh_sel, det_sel, sem):
    wid = lax.axis_index("s") * 2 + lax.axis_index("c")
    base = wid * _NCH
    pltpu.sync_copy(idx_hbm.at[pl.ds(base, _NCH)], idx_v)
    pltpu.sync_copy(hi3_hbm.at[pl.ds(base, _NCH)], hi3_v)
    copies = []
    for j in range(_NCH):
        copies.append(pltpu.async_copy(inh_hbm.at[hi3_v.at[j]], inh_rows.at[j], sem))
        copies.append(pltpu.async_copy(det_hbm.at[hi3_v.at[j]], det_rows.at[j], sem))
    for c in copies:
        c.wait()
    iotas = [lax.iota(jnp.int32, _L) + (_L * t) for t in range(_CH // _L)]
    for j in range(_NCH):
        for t in range(_CH // _L):
            o = _L * t
            lo = lax.bitwise_and(idx_v[j, pl.ds(o, _L)], 7)
            row = iotas[t]
            inh_sel[j, pl.ds(o, _L)] = plsc.load_gather(
                inh_rows.at[j], [row, lo])
            det_sel[j, pl.ds(o, _L)] = plsc.load_gather(
                det_rows.at[j], [row, lo])
    pltpu.sync_copy(inh_sel, inh_out.at[pl.ds(base, _NCH)])
    pltpu.sync_copy(det_sel, det_out.at[pl.ds(base, _NCH)])


def kernel(piano_model, embedding, inharm_embedding, detune_embedding):
    idx = piano_model.astype(jnp.int32)
    idx2d = idx.reshape(_NW * _NCH, _CH)
    hi1_2d = (idx >> 1).reshape(_NW * _NCH, _CH)
    hi3_2d = (idx >> 3).reshape(_NW * _NCH, _CH)
    zpairs = _gather_z(hi1_2d, embedding.reshape(-1, 2 * _D))
    inh, det = _gather_small(
        idx2d, hi3_2d,
        inharm_embedding.reshape(-1, _W),
        detune_embedding.reshape(-1, _W))
    zp = zpairs.reshape(_B, 2, _D)
    odd = (idx & 1).astype(bool)
    z = jnp.where(odd[:, None], zp[:, 1, :], zp[:, 0, :])
    return (z.reshape(_B, 1, _D),
            inh.reshape(_B, 1, 1),
            det.reshape(_B, 1, 1))
